# baseline (speedup 1.0000x reference)
import functools

import jax
import jax.numpy as jnp
from jax import lax
from jax.experimental import pallas as pl
from jax.experimental.pallas import tpu as pltpu
from jax.experimental.pallas import tpu_sc as plsc

D = 32
DB = D // 8
NC, NS = 2, 16
NW = NC * NS
BATCH = 128
STEPS = 200
GRP = 8
NGRP = STEPS // GRP
NOB = 4

_mesh = plsc.VectorSubcoreMesh(core_axis_name="c", subcore_axis_name="s")


@functools.partial(
    pl.kernel,
    out_type=jax.ShapeDtypeStruct((STEPS, DB, NW, 8 * BATCH), jnp.float32),
    mesh=_mesh,
    compiler_params=pltpu.CompilerParams(
        use_tc_tiling_on_sc=False,
        needs_layout_passes=False,
        disable_bounds_checks=True,
    ),
    scratch_types=[
        pltpu.VMEM((NGRP, GRP, BATCH), jnp.int32),
        pltpu.VMEM((2, GRP, BATCH, D), jnp.float32),
        pltpu.VMEM((NOB, DB, 8 * BATCH), jnp.float32),
        pltpu.SemaphoreType.DMA,
        pltpu.SemaphoreType.DMA,
        pltpu.SemaphoreType.DMA,
        pltpu.SemaphoreType.DMA,
        pltpu.SemaphoreType.DMA,
        pltpu.SemaphoreType.DMA,
        pltpu.SemaphoreType.DMA,
    ],
)
def _emb_gather(idx_hbm, table_hbm, out_hbm, idx_v, rows_v, trows_v,
                isem, gsem0, gsem1, osem0, osem1, osem2, osem3):
    wid = lax.axis_index("s") * NC + lax.axis_index("c")
    for sb in range(NGRP):
        pltpu.async_copy(idx_hbm.at[sb, wid], idx_v.at[sb], isem)
    for sb in range(NGRP):
        pltpu.make_async_copy(idx_hbm.at[sb, wid], idx_v.at[sb], isem).wait()

    gsems = (gsem0, gsem1)
    osems = (osem0, osem1, osem2, osem3)
    iota = lax.iota(jnp.int32, 16)
    rowvec = tuple(h * 2 + lax.shift_right_logical(iota, 3) for h in range(2))
    colbase = lax.rem(iota, 8) * 128

    def fire_group(g, b):
        for k in range(GRP):
            pltpu.async_copy(
                table_hbm.at[idx_v.at[g, k]], rows_v.at[b, k], gsems[b]
            )

    def drain_gather(b, k):
        pltpu.make_async_copy(
            table_hbm.at[pl.ds(0, BATCH)], rows_v.at[b, k], gsems[b]
        ).wait()

    def transpose(b, k, tb):
        @plsc.parallel_loop(0, BATCH, step=16, unroll=2)
        def jbody(j0):
            for u in range(16):
                jrow = j0 + u
                for h in range(2):
                    v = rows_v[b, k, jrow, pl.ds(h * 16, 16)]
                    plsc.store_scatter(
                        trows_v.at[tb], [rowvec[h], colbase + jrow], v
                    )

    def start_out(s, tb):
        pltpu.async_copy(
            trows_v.at[tb], out_hbm.at[s, :, wid], osems[tb]
        )

    def drain_out(tb):
        pltpu.make_async_copy(
            trows_v.at[tb], out_hbm.at[0, :, 0], osems[tb]
        ).wait()

    def group(g, b, drain_outs, fire_next):
        for k in range(GRP):
            drain_gather(b, k)
            if drain_outs or k >= NOB:
                drain_out(k % NOB)
            transpose(b, k, k % NOB)
            start_out(g * GRP + k, k % NOB)
        if fire_next is None:
            @pl.when(g + 2 < NGRP)
            def _():
                fire_group(g + 2, b)
        elif fire_next:
            fire_group(g + 2, b)

    fire_group(0, 0)
    fire_group(1, 1)
    group(0, 0, False, True)

    def loop_body(t, carry):
        group(2 * t + 1, 1, True, None)
        group(2 * t + 2, 0, True, None)
        return carry

    lax.fori_loop(0, (NGRP - 1) // 2, loop_body, 0)

    for tb in range(NOB):
        drain_out(tb)


def kernel(token_ids, embeddings):
    b, s = token_ids.shape
    idx = token_ids.T.reshape(NGRP, GRP, NW, BATCH).transpose(0, 2, 1, 3)
    out4 = _emb_gather(idx, embeddings)
    out5 = out4.reshape(STEPS, DB, NW, 8, BATCH)
    return out5.transpose(2, 4, 0, 1, 3).reshape(b, s, D)


# speedup vs baseline: 1.0093x; 1.0093x over previous
import functools

import jax
import jax.numpy as jnp
from jax import lax
from jax.experimental import pallas as pl
from jax.experimental.pallas import tpu as pltpu
from jax.experimental.pallas import tpu_sc as plsc

D = 32
DB = D // 8
NC, NS = 2, 16
NW = NC * NS
BATCH = 128
STEPS = 200
GRP = 8
NGRP = STEPS // GRP
NOB = 4

_mesh = plsc.VectorSubcoreMesh(core_axis_name="c", subcore_axis_name="s")


@functools.partial(
    pl.kernel,
    out_type=jax.ShapeDtypeStruct((STEPS, DB, NW, 8 * BATCH), jnp.float32),
    mesh=_mesh,
    compiler_params=pltpu.CompilerParams(
        use_tc_tiling_on_sc=False,
        needs_layout_passes=False,
        disable_bounds_checks=True,
    ),
    scratch_types=[
        pltpu.VMEM((NGRP, GRP, BATCH), jnp.int32),
        pltpu.VMEM((2, GRP, BATCH, D), jnp.float32),
        pltpu.VMEM((NOB, DB, 8 * BATCH), jnp.float32),
        pltpu.SemaphoreType.DMA,
        pltpu.SemaphoreType.DMA,
        pltpu.SemaphoreType.DMA,
        pltpu.SemaphoreType.DMA,
        pltpu.SemaphoreType.DMA,
        pltpu.SemaphoreType.DMA,
        pltpu.SemaphoreType.DMA,
    ],
)
def _emb_gather(idx_hbm, table_hbm, out_hbm, idx_v, rows_v, trows_v,
                isem, gsem0, gsem1, osem0, osem1, osem2, osem3):
    wid = lax.axis_index("s") * NC + lax.axis_index("c")
    for sb in range(NGRP):
        pltpu.async_copy(idx_hbm.at[sb, wid], idx_v.at[sb], isem)
    for sb in range(NGRP):
        pltpu.make_async_copy(idx_hbm.at[sb, wid], idx_v.at[sb], isem).wait()

    gsems = (gsem0, gsem1)
    osems = (osem0, osem1, osem2, osem3)
    iota = lax.iota(jnp.int32, 16)
    rowvec = tuple(h * 2 + lax.shift_right_logical(iota, 3) for h in range(2))
    colbase = lax.rem(iota, 8) * 128

    def fire_group(g, b):
        for k in range(GRP):
            pltpu.async_copy(
                table_hbm.at[idx_v.at[g, k]], rows_v.at[b, k], gsems[b]
            )

    def drain_gather(b, k):
        pltpu.make_async_copy(
            table_hbm.at[pl.ds(0, BATCH)], rows_v.at[b, k], gsems[b]
        ).wait()

    def transpose(b, k, tb):
        @plsc.parallel_loop(0, BATCH, step=8, unroll=4)
        def jbody(j0):
            for u in range(8):
                jrow = j0 + u
                for h in range(2):
                    v = rows_v[b, k, jrow, pl.ds(h * 16, 16)]
                    plsc.store_scatter(
                        trows_v.at[tb], [rowvec[h], colbase + jrow], v
                    )

    def start_out(s, tb):
        pltpu.async_copy(
            trows_v.at[tb], out_hbm.at[s, :, wid], osems[tb]
        )

    def drain_out(tb):
        pltpu.make_async_copy(
            trows_v.at[tb], out_hbm.at[0, :, 0], osems[tb]
        ).wait()

    def group(g, b, drain_outs, fire_next):
        for k in range(GRP):
            drain_gather(b, k)
            if drain_outs or k >= NOB:
                drain_out(k % NOB)
            transpose(b, k, k % NOB)
            start_out(g * GRP + k, k % NOB)
        if fire_next is None:
            @pl.when(g + 2 < NGRP)
            def _():
                fire_group(g + 2, b)
        elif fire_next:
            fire_group(g + 2, b)

    fire_group(0, 0)
    fire_group(1, 1)
    group(0, 0, False, True)

    def loop_body(t, carry):
        group(2 * t + 1, 1, True, None)
        group(2 * t + 2, 0, True, None)
        return carry

    lax.fori_loop(0, (NGRP - 1) // 2, loop_body, 0)

    for tb in range(NOB):
        drain_out(tb)


def kernel(token_ids, embeddings):
    b, s = token_ids.shape
    idx = token_ids.T.reshape(NGRP, GRP, NW, BATCH).transpose(0, 2, 1, 3)
    out4 = _emb_gather(idx, embeddings)
    out5 = out4.reshape(STEPS, DB, NW, 8, BATCH)
    return out5.transpose(2, 4, 0, 1, 3).reshape(b, s, D)
